# baseline (speedup 1.0000x reference)
import jax
import jax.numpy as jnp
from jax.experimental import pallas as pl
from jax.experimental.pallas import tpu as pltpu

_N = 16
_ROWS = 27648 // _N
_COLS = 1024


def _copy_body(in_ref, out_ref, sems):
    for i in range(_N):
        pltpu.make_async_copy(in_ref.at[i], out_ref.at[i], sems.at[i]).start()
    for i in range(_N):
        pltpu.make_async_copy(in_ref.at[i], out_ref.at[i], sems.at[i]).wait()


def kernel(_input):
    x = _input.reshape(_N, _ROWS, _COLS)
    out = pl.pallas_call(
        _copy_body,
        in_specs=[pl.BlockSpec(memory_space=pl.ANY)],
        out_specs=pl.BlockSpec(memory_space=pl.ANY),
        out_shape=jax.ShapeDtypeStruct((_N, _ROWS, _COLS), _input.dtype),
        scratch_shapes=[pltpu.SemaphoreType.DMA((_N,))],
    )(x)
    return out.reshape(_input.shape)


# speedup vs baseline: 2.7758x; 2.7758x over previous
import jax
import jax.numpy as jnp
from jax.experimental import pallas as pl
from jax.experimental.pallas import tpu as pltpu

_C = 32
_B = 16
_ROWS = 27648 // _C
_COLS = 1024


def _copy_body(in_ref, out_ref, vmem, in_sems, out_sems):
    for i in range(_B):
        pltpu.make_async_copy(in_ref.at[i], vmem.at[i], in_sems.at[i]).start()
    for i in range(_C):
        b = i % _B
        pltpu.make_async_copy(in_ref.at[i], vmem.at[b], in_sems.at[i]).wait()
        pltpu.make_async_copy(vmem.at[b], out_ref.at[i], out_sems.at[i]).start()
        j = i + _B
        if j < _C:
            pltpu.make_async_copy(vmem.at[b], out_ref.at[i], out_sems.at[i]).wait()
            pltpu.make_async_copy(in_ref.at[j], vmem.at[b], in_sems.at[j]).start()
    for i in range(_C - _B, _C):
        b = i % _B
        pltpu.make_async_copy(vmem.at[b], out_ref.at[i], out_sems.at[i]).wait()


def kernel(_input):
    x = _input.reshape(_C, _ROWS, _COLS)
    out = pl.pallas_call(
        _copy_body,
        in_specs=[pl.BlockSpec(memory_space=pl.ANY)],
        out_specs=pl.BlockSpec(memory_space=pl.ANY),
        out_shape=jax.ShapeDtypeStruct((_C, _ROWS, _COLS), _input.dtype),
        scratch_shapes=[
            pltpu.VMEM((_B, _ROWS, _COLS), jnp.float32),
            pltpu.SemaphoreType.DMA((_C,)),
            pltpu.SemaphoreType.DMA((_C,)),
        ],
    )(x)
    return out.reshape(_input.shape)


# speedup vs baseline: 2.7811x; 1.0019x over previous
import functools

import jax
import jax.numpy as jnp
from jax import lax
from jax.experimental import pallas as pl
from jax.experimental.pallas import tpu as pltpu
from jax.experimental.pallas import tpu_sc as plsc

_TOTAL = 64 * 768 * 24 * 24
_NW = 32
_PER_W = _TOTAL // _NW
_CHUNK = 32768
_NCHUNK = _PER_W // _CHUNK

_MESH = plsc.VectorSubcoreMesh(core_axis_name="c", subcore_axis_name="s")


@functools.partial(
    pl.kernel,
    mesh=_MESH,
    out_type=jax.ShapeDtypeStruct((_TOTAL,), jnp.float32),
    scratch_types=[
        pltpu.VMEM((_CHUNK,), jnp.float32),
        pltpu.VMEM((_CHUNK,), jnp.float32),
        pltpu.SemaphoreType.DMA,
        pltpu.SemaphoreType.DMA,
        pltpu.SemaphoreType.DMA,
        pltpu.SemaphoreType.DMA,
    ],
)
def _sc_copy(in_hbm, out_hbm, buf0, buf1, si0, si1, so0, so1):
    wid = lax.axis_index("s") * 2 + lax.axis_index("c")
    base = wid * _PER_W
    bufs = (buf0, buf1)
    isems = (si0, si1)
    osems = (so0, so1)

    def in_copy(c, b):
        return pltpu.make_async_copy(
            in_hbm.at[pl.ds(base + c * _CHUNK, _CHUNK)], bufs[b], isems[b])

    def out_copy(c, b):
        return pltpu.make_async_copy(
            bufs[b], out_hbm.at[pl.ds(base + c * _CHUNK, _CHUNK)], osems[b])

    in_copy(0, 0).start()
    for c in range(_NCHUNK):
        b = c & 1
        if c + 1 < _NCHUNK:
            nb = (c + 1) & 1
            if c >= 1:
                out_copy(c - 1, nb).wait()
            in_copy(c + 1, nb).start()
        in_copy(c, b).wait()
        out_copy(c, b).start()
    out_copy(_NCHUNK - 2, _NCHUNK & 1).wait()
    out_copy(_NCHUNK - 1, (_NCHUNK - 1) & 1).wait()


def kernel(_input):
    out = _sc_copy(_input.reshape(_TOTAL))
    return out.reshape(_input.shape)


# speedup vs baseline: 5.0575x; 1.8186x over previous
import jax
import jax.numpy as jnp
from jax.experimental import pallas as pl
from jax.experimental.pallas import tpu as pltpu


def _copy_body(in_ref, out_ref):
    out_ref[...] = in_ref[...]


def kernel(_input):
    n, c, h, w = _input.shape
    out = pl.pallas_call(
        _copy_body,
        grid=(n,),
        in_specs=[pl.BlockSpec((1, c, h, w), lambda i: (i, 0, 0, 0))],
        out_specs=pl.BlockSpec((1, c, h, w), lambda i: (i, 0, 0, 0)),
        out_shape=jax.ShapeDtypeStruct(_input.shape, _input.dtype),
        compiler_params=pltpu.CompilerParams(
            dimension_semantics=("arbitrary",),
        ),
    )(_input)
    return out
